# baseline (speedup 1.0000x reference)
import math

import jax
import jax.numpy as jnp
from jax.experimental import pallas as pl
from jax.experimental.pallas import tpu as pltpu

N = 10000
E = 160000
NDIM = 256
H = 8
DK = NDIM // H
DFF = 4 * NDIM

ROW_BLK = 1000


def _qkv_body(x_ref, w_ref, bq_ref, out_ref):
    x = x_ref[...]
    w = w_ref[...]
    acc = jnp.dot(x, w, preferred_element_type=jnp.float32)
    out_ref[...] = acc + bq_ref[...]


def _qkv(x, wqkv, bqkv):
    grid = (N // ROW_BLK,)
    return pl.pallas_call(
        _qkv_body,
        grid=grid,
        in_specs=[
            pl.BlockSpec((ROW_BLK, NDIM), lambda i: (i, 0)),
            pl.BlockSpec((NDIM, 3 * NDIM), lambda i: (0, 0)),
            pl.BlockSpec((1, 3 * NDIM), lambda i: (0, 0)),
        ],
        out_specs=pl.BlockSpec((ROW_BLK, 3 * NDIM), lambda i: (i, 0)),
        out_shape=jax.ShapeDtypeStruct((N, 3 * NDIM), jnp.float32),
    )(x, wqkv, bqkv)


def _ln(h, g, b, eps=1e-5):
    m = jnp.mean(h, axis=-1, keepdims=True)
    c = h - m
    v = jnp.mean(c * c, axis=-1, keepdims=True)
    return c * jax.lax.rsqrt(v + eps) * g + b


def _post_body(x_ref, wv_ref, zb_ref, wo_ref, bo_ref, lng_ref, lnb_ref,
               w1_ref, b1_ref, w2_ref, b2_ref, ln2g_ref, ln2b_ref, out_ref):
    o = wv_ref[...] / (zb_ref[...] + 1e-12)
    x = x_ref[...]
    h = _ln(x + jnp.dot(o, wo_ref[...], preferred_element_type=jnp.float32)
            + bo_ref[...], lng_ref[...], lnb_ref[...])
    f = jnp.maximum(jnp.dot(h, w1_ref[...], preferred_element_type=jnp.float32)
                    + b1_ref[...], 0.0)
    out = _ln(h + jnp.dot(f, w2_ref[...], preferred_element_type=jnp.float32)
              + b2_ref[...], ln2g_ref[...], ln2b_ref[...])
    out_ref[...] = out


def _post(x, wv, zb, Wo, bo, ln_g, ln_b, W1, b1, W2, b2, ln2_g, ln2_b):
    grid = (N // ROW_BLK,)
    row = lambda i: (i, 0)
    fixed = lambda i: (0, 0)
    return pl.pallas_call(
        _post_body,
        grid=grid,
        in_specs=[
            pl.BlockSpec((ROW_BLK, NDIM), row),
            pl.BlockSpec((ROW_BLK, NDIM), row),
            pl.BlockSpec((ROW_BLK, NDIM), row),
            pl.BlockSpec((NDIM, NDIM), fixed),
            pl.BlockSpec((1, NDIM), fixed),
            pl.BlockSpec((1, NDIM), fixed),
            pl.BlockSpec((1, NDIM), fixed),
            pl.BlockSpec((NDIM, DFF), fixed),
            pl.BlockSpec((1, DFF), fixed),
            pl.BlockSpec((DFF, NDIM), fixed),
            pl.BlockSpec((1, NDIM), fixed),
            pl.BlockSpec((1, NDIM), fixed),
            pl.BlockSpec((1, NDIM), fixed),
        ],
        out_specs=pl.BlockSpec((ROW_BLK, NDIM), row),
        out_shape=jax.ShapeDtypeStruct((N, NDIM), jnp.float32),
    )(x, wv, zb, Wo, bo.reshape(1, NDIM), ln_g.reshape(1, NDIM),
      ln_b.reshape(1, NDIM), W1, b1.reshape(1, DFF), W2, b2.reshape(1, NDIM),
      ln2_g.reshape(1, NDIM), ln2_b.reshape(1, NDIM))


def kernel(x, edge_index, Wq, bq, Wk, Wv, Wo, bo, ln_g, ln_b, W1, b1, W2, b2,
           ln2_g, ln2_b):
    wqkv = jnp.concatenate([Wq, Wk, Wv], axis=1)
    bqkv = jnp.concatenate([bq, jnp.zeros((2 * NDIM,), jnp.float32)]).reshape(1, 3 * NDIM)
    qkv = _qkv(x, wqkv, bqkv)
    q = qkv[:, :NDIM].reshape(N, H, DK)
    k = qkv[:, NDIM:2 * NDIM].reshape(N, H, DK)
    v = qkv[:, 2 * NDIM:].reshape(N, H, DK)

    src = edge_index[0]
    dst = edge_index[1]
    score = jnp.sum(k[src] * q[dst], axis=-1)
    score = jnp.exp(jnp.clip(score / math.sqrt(DK), -5.0, 5.0))
    wv = jax.ops.segment_sum(v[src] * score[:, :, None], dst, num_segments=N)
    z = jax.ops.segment_sum(score, dst, num_segments=N)

    wv2 = wv.reshape(N, NDIM)
    zb = jnp.broadcast_to(z[:, :, None], (N, H, DK)).reshape(N, NDIM)
    return _post(x, wv2, zb, Wo, bo, ln_g, ln_b, W1, b1, W2, b2, ln2_g, ln2_b)


# speedup vs baseline: 11.2027x; 11.2027x over previous
import functools
import math

import jax
import jax.numpy as jnp
from jax import lax
from jax.experimental import pallas as pl
from jax.experimental.pallas import tpu as pltpu
from jax.experimental.pallas import tpu_sc as plsc

N = 10000
E = 160000
NDIM = 256
H = 8
DK = NDIM // H
DFF = 4 * NDIM

NC = 2
NS = 16
NW = NC * NS

ROW_BLK = 1000
EBLK = 1000
UW = 144

CA = 128
EPW = E // NW
NA_FULL = EPW // CA
A_TAIL_BASE = EPW - CA

EPT = E // NS
NB_FULL = EPT // CA
B_TAIL = EPT - NB_FULL * CA
RPT = N // NS

_mesh = plsc.VectorSubcoreMesh(
    core_axis_name="c", subcore_axis_name="s", num_cores=NC, num_subcores=NS)
_sc_params = pltpu.CompilerParams(use_tc_tiling_on_sc=False)


def _qkv_body(x_ref, w_ref, bq_ref, q_ref, kv_ref):
    acc = jnp.dot(x_ref[...], w_ref[...], preferred_element_type=jnp.float32)
    q_ref[...] = acc[:, :NDIM] + bq_ref[...]
    kv_ref[...] = acc[:, NDIM:]


def _qkv(x, wqkv, bq):
    return pl.pallas_call(
        _qkv_body,
        grid=(N // ROW_BLK,),
        in_specs=[
            pl.BlockSpec((ROW_BLK, NDIM), lambda i: (i, 0)),
            pl.BlockSpec((NDIM, 3 * NDIM), lambda i: (0, 0)),
            pl.BlockSpec((1, NDIM), lambda i: (0, 0)),
        ],
        out_specs=[
            pl.BlockSpec((ROW_BLK, NDIM), lambda i: (i, 0)),
            pl.BlockSpec((ROW_BLK, 2 * NDIM), lambda i: (i, 0)),
        ],
        out_shape=[
            jax.ShapeDtypeStruct((N, NDIM), jnp.float32),
            jax.ShapeDtypeStruct((N, 2 * NDIM), jnp.float32),
        ],
    )(x, wqkv, bq.reshape(1, NDIM))


def _gather_body(kv_hbm, q_hbm, src_hbm, dst_hbm, kvg_hbm, qg_hbm,
                 sidx, didx, kvbuf, qbuf, sem1, sem2):
    c = lax.axis_index("c")
    s = lax.axis_index("s")
    base0 = (s * NC + c) * EPW

    def chunk(base):
        pltpu.sync_copy(src_hbm.at[pl.ds(base, CA)], sidx)
        pltpu.sync_copy(dst_hbm.at[pl.ds(base, CA)], didx)
        cp1 = pltpu.async_copy(kv_hbm.at[sidx], kvbuf, sem1)
        cp2 = pltpu.async_copy(q_hbm.at[didx], qbuf, sem2)
        cp1.wait()
        cp2.wait()
        pltpu.sync_copy(kvbuf, kvg_hbm.at[pl.ds(base, CA)])
        pltpu.sync_copy(qbuf, qg_hbm.at[pl.ds(base, CA)])

    def body(i, carry):
        chunk(base0 + i * CA)
        return carry

    lax.fori_loop(0, NA_FULL, body, 0)
    chunk(base0 + A_TAIL_BASE)


def _gather(kv, q, src, dst):
    f = pl.kernel(
        _gather_body,
        out_type=[
            jax.ShapeDtypeStruct((E, 2 * NDIM), jnp.float32),
            jax.ShapeDtypeStruct((E, NDIM), jnp.float32),
        ],
        mesh=_mesh,
        scratch_types=[
            pltpu.VMEM((CA,), jnp.int32),
            pltpu.VMEM((CA,), jnp.int32),
            pltpu.VMEM((CA, 2 * NDIM), jnp.float32),
            pltpu.VMEM((CA, NDIM), jnp.float32),
            pltpu.SemaphoreType.DMA,
            pltpu.SemaphoreType.DMA,
        ],
        compiler_params=_sc_params,
    )
    return f(kv, q, src, dst)


def _score_body(kvg_ref, qg_ref, uL_ref, uR_ref):
    kv = kvg_ref[...]
    qg = qg_ref[...]
    kg = kv[:, :NDIM]
    vg = kv[:, NDIM:]
    m = (lax.broadcasted_iota(jnp.int32, (NDIM, H), 0) // DK
         == lax.broadcasted_iota(jnp.int32, (NDIM, H), 1)).astype(jnp.float32)
    m2 = (lax.broadcasted_iota(jnp.int32, (H, NDIM), 1) // DK
          == lax.broadcasted_iota(jnp.int32, (H, NDIM), 0)).astype(jnp.float32)
    sc = jnp.dot(kg * qg, m, preferred_element_type=jnp.float32)
    sc = jnp.exp(jnp.clip(sc * (1.0 / math.sqrt(DK)), -5.0, 5.0))
    sb = jnp.dot(sc, m2, preferred_element_type=jnp.float32)
    wvg = vg * sb
    zpad = jnp.zeros((EBLK, UW - NDIM // 2 - H // 2), jnp.float32)
    uL_ref[...] = jnp.concatenate([wvg[:, :NDIM // 2], sc[:, :H // 2], zpad], axis=1)
    uR_ref[...] = jnp.concatenate([wvg[:, NDIM // 2:], sc[:, H // 2:], zpad], axis=1)


def _score(kvg, qg):
    return pl.pallas_call(
        _score_body,
        grid=(E // EBLK,),
        in_specs=[
            pl.BlockSpec((EBLK, 2 * NDIM), lambda i: (i, 0)),
            pl.BlockSpec((EBLK, NDIM), lambda i: (i, 0)),
        ],
        out_specs=[
            pl.BlockSpec((EBLK, UW), lambda i: (i, 0)),
            pl.BlockSpec((EBLK, UW), lambda i: (i, 0)),
        ],
        out_shape=[
            jax.ShapeDtypeStruct((E, UW), jnp.float32),
            jax.ShapeDtypeStruct((E, UW), jnp.float32),
        ],
    )(kvg, qg)


def _scatter_body(uL_hbm, uR_hbm, dst_hbm, zeros_hbm, out_hbm,
                  didx, didx_t, ubuf, ubuf_t, accum):
    c = lax.axis_index("c")
    s = lax.axis_index("s")
    rbase = s * RPT
    pltpu.sync_copy(zeros_hbm, accum.at[pl.ds(rbase, RPT)])
    plsc.subcore_barrier()
    ebase = s * EPT

    def do(u_hbm):
        def body(i, carry):
            b = ebase + i * CA
            pltpu.sync_copy(dst_hbm.at[pl.ds(b, CA)], didx)
            pltpu.sync_copy(u_hbm.at[pl.ds(b, CA)], ubuf)
            pltpu.sync_copy(ubuf, accum.at[didx], add=True)
            return carry

        lax.fori_loop(0, NB_FULL, body, 0)
        b = ebase + NB_FULL * CA
        pltpu.sync_copy(dst_hbm.at[pl.ds(b, B_TAIL)], didx_t)
        pltpu.sync_copy(u_hbm.at[pl.ds(b, B_TAIL)], ubuf_t)
        pltpu.sync_copy(ubuf_t, accum.at[didx_t], add=True)

    @pl.when(c == 0)
    def _():
        do(uL_hbm)

    @pl.when(c == 1)
    def _():
        do(uR_hbm)

    plsc.subcore_barrier()

    @pl.when(c == 0)
    def _():
        pltpu.sync_copy(accum.at[pl.ds(rbase, RPT)],
                        out_hbm.at[0, pl.ds(rbase, RPT)])

    @pl.when(c == 1)
    def _():
        pltpu.sync_copy(accum.at[pl.ds(rbase, RPT)],
                        out_hbm.at[1, pl.ds(rbase, RPT)])


def _scatter(uL, uR, dst, zeros):
    f = pl.kernel(
        _scatter_body,
        out_type=jax.ShapeDtypeStruct((2, N, UW), jnp.float32),
        mesh=_mesh,
        scratch_types=[
            pltpu.VMEM((CA,), jnp.int32),
            pltpu.VMEM((B_TAIL,), jnp.int32),
            pltpu.VMEM((CA, UW), jnp.float32),
            pltpu.VMEM((B_TAIL, UW), jnp.float32),
            pltpu.VMEM_SHARED((N, UW), jnp.float32),
        ],
        compiler_params=_sc_params,
    )
    return f(uL, uR, dst, zeros)


def _ln(h, g, b, eps=1e-5):
    m = jnp.mean(h, axis=-1, keepdims=True)
    cc = h - m
    v = jnp.mean(cc * cc, axis=-1, keepdims=True)
    return cc * lax.rsqrt(v + eps) * g + b


def _post_body(x_ref, wv_ref, zb_ref, wo_ref, bo_ref, lng_ref, lnb_ref,
               w1_ref, b1_ref, w2_ref, b2_ref, ln2g_ref, ln2b_ref, out_ref):
    o = wv_ref[...] / (zb_ref[...] + 1e-12)
    x = x_ref[...]
    h = _ln(x + jnp.dot(o, wo_ref[...], preferred_element_type=jnp.float32)
            + bo_ref[...], lng_ref[...], lnb_ref[...])
    f = jnp.maximum(jnp.dot(h, w1_ref[...], preferred_element_type=jnp.float32)
                    + b1_ref[...], 0.0)
    out_ref[...] = _ln(h + jnp.dot(f, w2_ref[...], preferred_element_type=jnp.float32)
                       + b2_ref[...], ln2g_ref[...], ln2b_ref[...])


def _post(x, wv, zb, Wo, bo, ln_g, ln_b, W1, b1, W2, b2, ln2_g, ln2_b):
    row = lambda i: (i, 0)
    fixed = lambda i: (0, 0)
    return pl.pallas_call(
        _post_body,
        grid=(N // ROW_BLK,),
        in_specs=[
            pl.BlockSpec((ROW_BLK, NDIM), row),
            pl.BlockSpec((ROW_BLK, NDIM), row),
            pl.BlockSpec((ROW_BLK, NDIM), row),
            pl.BlockSpec((NDIM, NDIM), fixed),
            pl.BlockSpec((1, NDIM), fixed),
            pl.BlockSpec((1, NDIM), fixed),
            pl.BlockSpec((1, NDIM), fixed),
            pl.BlockSpec((NDIM, DFF), fixed),
            pl.BlockSpec((1, DFF), fixed),
            pl.BlockSpec((DFF, NDIM), fixed),
            pl.BlockSpec((1, NDIM), fixed),
            pl.BlockSpec((1, NDIM), fixed),
            pl.BlockSpec((1, NDIM), fixed),
        ],
        out_specs=pl.BlockSpec((ROW_BLK, NDIM), row),
        out_shape=jax.ShapeDtypeStruct((N, NDIM), jnp.float32),
    )(x, wv, zb, Wo, bo.reshape(1, NDIM), ln_g.reshape(1, NDIM),
      ln_b.reshape(1, NDIM), W1, b1.reshape(1, DFF), W2, b2.reshape(1, NDIM),
      ln2_g.reshape(1, NDIM), ln2_b.reshape(1, NDIM))


def kernel(x, edge_index, Wq, bq, Wk, Wv, Wo, bo, ln_g, ln_b, W1, b1, W2, b2,
           ln2_g, ln2_b):
    wqkv = jnp.concatenate([Wq, Wk, Wv], axis=1)
    q, kv = _qkv(x, wqkv, bq)

    src = edge_index[0]
    dst = edge_index[1]
    kvg, qg = _gather(kv, q, src, dst)
    uL, uR = _score(kvg, qg)
    zeros = jnp.zeros((RPT, UW), jnp.float32)
    agg = _scatter(uL, uR, dst, zeros)

    wv = jnp.concatenate([agg[0, :, :NDIM // 2], agg[1, :, :NDIM // 2]], axis=1)
    z = jnp.concatenate([agg[0, :, NDIM // 2:NDIM // 2 + H // 2],
                         agg[1, :, NDIM // 2:NDIM // 2 + H // 2]], axis=1)
    zb = jnp.broadcast_to(z[:, :, None], (N, H, DK)).reshape(N, NDIM)
    return _post(x, wv, zb, Wo, bo, ln_g, ln_b, W1, b1, W2, b2, ln2_g, ln2_b)
